# baseline (device time: 19529 ns/iter reference)
import jax
import jax.numpy as jnp
from jax import lax
from jax.experimental import pallas as pl
from jax.experimental.pallas import tpu as pltpu

N_DEV = 4
C_GLOBAL = 2048
EPS = 1e-5
NCHUNK = 8


def kernel(x, t_emb, W_scale, W_shift):
    b, s, c_loc = x.shape
    cs = s // NCHUNK

    def body(x_hbm, t_ref, ws_ref, wsh_ref, out_hbm,
             xv_ref, ob_ref, own_ref, comm_ref,
             in_sems, out_sems, send_sems, recv_sems):
        my = lax.axis_index("i")

        in_dmas = []
        for k in range(NCHUNK):
            dma = pltpu.make_async_copy(
                x_hbm.at[:, pl.ds(k * cs, cs), :],
                xv_ref.at[:, pl.ds(k * cs, cs), :],
                in_sems.at[k],
            )
            dma.start()
            in_dmas.append(dma)

        barrier_sem = pltpu.get_barrier_semaphore()
        for d in (1, 2, 3):
            pl.semaphore_signal(
                barrier_sem, inc=1,
                device_id=((my + d) % N_DEV,),
                device_id_type=pl.DeviceIdType.MESH,
            )
        pl.semaphore_wait(barrier_sem, 3)

        for k in range(NCHUNK):
            in_dmas[k].wait()
            xk = xv_ref[:, pl.ds(k * cs, cs), :]
            own_ref[0:b, pl.ds(k * cs, cs)] = jnp.sum(xk, axis=-1)
            own_ref[b:2 * b, pl.ds(k * cs, cs)] = jnp.sum(xk * xk, axis=-1)

        rdmas = []
        for d in (1, 2, 3):
            rdma = pltpu.make_async_remote_copy(
                src_ref=own_ref,
                dst_ref=comm_ref.at[3 - d],
                send_sem=send_sems.at[d - 1],
                recv_sem=recv_sems.at[3 - d],
                device_id=((my + d) % N_DEV,),
                device_id_type=pl.DeviceIdType.MESH,
            )
            rdma.start()
            rdmas.append(rdma)

        t = t_ref[...]
        scale = jnp.dot(t, ws_ref[...], preferred_element_type=jnp.float32)
        shift = jnp.dot(t, wsh_ref[...], preferred_element_type=jnp.float32)
        sc_b = (1.0 + scale).astype(jnp.bfloat16)[:, None, :]
        sh_b = shift.astype(jnp.bfloat16)[:, None, :]

        for rdma in rdmas:
            rdma.wait_recv()

        tot = (own_ref[...] + comm_ref[0] + comm_ref[1] + comm_ref[2])
        mean = tot[:b] / C_GLOBAL
        var = tot[b:] / C_GLOBAL - mean * mean
        inv = lax.rsqrt(var + EPS)
        mean_b = mean.astype(jnp.bfloat16)
        inv_b = inv.astype(jnp.bfloat16)

        out_dmas = []
        for k in range(NCHUNK):
            sl = pl.ds(k * cs, cs)
            xb = xv_ref[:, sl, :].astype(jnp.bfloat16)
            mk = mean_b[:, k * cs:(k + 1) * cs, None]
            ik = inv_b[:, k * cs:(k + 1) * cs, None]
            ob_ref[:, sl, :] = ((xb - mk) * ik) * sc_b + sh_b
            dma = pltpu.make_async_copy(
                ob_ref.at[:, sl, :], out_hbm.at[:, sl, :], out_sems.at[k]
            )
            dma.start()
            out_dmas.append(dma)

        for dma in out_dmas:
            dma.wait()
        for rdma in rdmas:
            rdma.wait_send()

    return pl.pallas_call(
        body,
        out_shape=jax.ShapeDtypeStruct((b, s, c_loc), jnp.bfloat16),
        in_specs=[
            pl.BlockSpec(memory_space=pl.ANY),
            pl.BlockSpec(memory_space=pltpu.VMEM),
            pl.BlockSpec(memory_space=pltpu.VMEM),
            pl.BlockSpec(memory_space=pltpu.VMEM),
        ],
        out_specs=pl.BlockSpec(memory_space=pl.ANY),
        scratch_shapes=[
            pltpu.VMEM((b, s, c_loc), jnp.float32),
            pltpu.VMEM((b, s, c_loc), jnp.bfloat16),
            pltpu.VMEM((2 * b, s), jnp.float32),
            pltpu.VMEM((3, 2 * b, s), jnp.float32),
            pltpu.SemaphoreType.DMA((NCHUNK,)),
            pltpu.SemaphoreType.DMA((NCHUNK,)),
            pltpu.SemaphoreType.DMA((3,)),
            pltpu.SemaphoreType.DMA((3,)),
        ],
        compiler_params=pltpu.CompilerParams(collective_id=0),
    )(x, t_emb, W_scale, W_shift)


# device time: 18401 ns/iter; 1.0613x vs baseline; 1.0613x over previous
import jax
import jax.numpy as jnp
from jax import lax
from jax.experimental import pallas as pl
from jax.experimental.pallas import tpu as pltpu

N_DEV = 4
C_GLOBAL = 2048
EPS = 1e-5
NCHUNK = 2


def kernel(x, t_emb, W_scale, W_shift):
    b, s, c_loc = x.shape
    cs = s // NCHUNK

    def body(x_hbm, t_ref, ws_ref, wsh_ref, out_hbm,
             xv_ref, ob_ref, own_ref, comm_ref,
             in_sems, out_sems, send_sems, recv_sems):
        my = lax.axis_index("i")

        in_dmas = []
        for k in range(NCHUNK):
            dma = pltpu.make_async_copy(
                x_hbm.at[:, pl.ds(k * cs, cs), :],
                xv_ref.at[:, pl.ds(k * cs, cs), :],
                in_sems.at[k],
            )
            dma.start()
            in_dmas.append(dma)

        barrier_sem = pltpu.get_barrier_semaphore()
        for d in (1, 2, 3):
            pl.semaphore_signal(
                barrier_sem, inc=1,
                device_id=((my + d) % N_DEV,),
                device_id_type=pl.DeviceIdType.MESH,
            )
        pl.semaphore_wait(barrier_sem, 3)

        for k in range(NCHUNK):
            in_dmas[k].wait()
            xk = xv_ref[:, pl.ds(k * cs, cs), :]
            own_ref[0:b, pl.ds(k * cs, cs)] = jnp.sum(xk, axis=-1)
            own_ref[b:2 * b, pl.ds(k * cs, cs)] = jnp.sum(xk * xk, axis=-1)

        rdmas = []
        for d in (1, 2, 3):
            rdma = pltpu.make_async_remote_copy(
                src_ref=own_ref,
                dst_ref=comm_ref.at[3 - d],
                send_sem=send_sems.at[d - 1],
                recv_sem=recv_sems.at[3 - d],
                device_id=((my + d) % N_DEV,),
                device_id_type=pl.DeviceIdType.MESH,
            )
            rdma.start()
            rdmas.append(rdma)

        t = t_ref[...]
        scale = jnp.dot(t, ws_ref[...], preferred_element_type=jnp.float32)
        shift = jnp.dot(t, wsh_ref[...], preferred_element_type=jnp.float32)
        sc_b = (1.0 + scale).astype(jnp.bfloat16)[:, None, :]
        sh_b = shift.astype(jnp.bfloat16)[:, None, :]

        for rdma in rdmas:
            rdma.wait_recv()

        tot = (own_ref[...] + comm_ref[0] + comm_ref[1] + comm_ref[2])
        mean = tot[:b] / C_GLOBAL
        var = tot[b:] / C_GLOBAL - mean * mean
        inv = lax.rsqrt(var + EPS)
        mean_b = mean.astype(jnp.bfloat16)
        inv_b = inv.astype(jnp.bfloat16)

        out_dmas = []
        for k in range(NCHUNK):
            sl = pl.ds(k * cs, cs)
            xb = xv_ref[:, sl, :].astype(jnp.bfloat16)
            mk = mean_b[:, k * cs:(k + 1) * cs, None]
            ik = inv_b[:, k * cs:(k + 1) * cs, None]
            ob_ref[:, sl, :] = ((xb - mk) * ik) * sc_b + sh_b
            dma = pltpu.make_async_copy(
                ob_ref.at[:, sl, :], out_hbm.at[:, sl, :], out_sems.at[k]
            )
            dma.start()
            out_dmas.append(dma)

        for dma in out_dmas:
            dma.wait()
        for rdma in rdmas:
            rdma.wait_send()

    return pl.pallas_call(
        body,
        out_shape=jax.ShapeDtypeStruct((b, s, c_loc), jnp.bfloat16),
        in_specs=[
            pl.BlockSpec(memory_space=pl.ANY),
            pl.BlockSpec(memory_space=pltpu.VMEM),
            pl.BlockSpec(memory_space=pltpu.VMEM),
            pl.BlockSpec(memory_space=pltpu.VMEM),
        ],
        out_specs=pl.BlockSpec(memory_space=pl.ANY),
        scratch_shapes=[
            pltpu.VMEM((b, s, c_loc), jnp.float32),
            pltpu.VMEM((b, s, c_loc), jnp.bfloat16),
            pltpu.VMEM((2 * b, s), jnp.float32),
            pltpu.VMEM((3, 2 * b, s), jnp.float32),
            pltpu.SemaphoreType.DMA((NCHUNK,)),
            pltpu.SemaphoreType.DMA((NCHUNK,)),
            pltpu.SemaphoreType.DMA((3,)),
            pltpu.SemaphoreType.DMA((3,)),
        ],
        compiler_params=pltpu.CompilerParams(collective_id=0),
    )(x, t_emb, W_scale, W_shift)


# device time: 16225 ns/iter; 1.2036x vs baseline; 1.1341x over previous
import jax
import jax.numpy as jnp
from jax import lax
from jax.experimental import pallas as pl
from jax.experimental.pallas import tpu as pltpu

N_DEV = 4
C_GLOBAL = 2048
EPS = 1e-5


def kernel(x, t_emb, W_scale, W_shift):
    b, s, c_loc = x.shape

    def body(x_ref, t_ref, ws_ref, wsh_ref, out_ref,
             own_ref, comm_ref, send_sems, recv_sems):
        my = lax.axis_index("i")

        barrier_sem = pltpu.get_barrier_semaphore()
        for d in (1, 2, 3):
            pl.semaphore_signal(
                barrier_sem, inc=1,
                device_id=((my + d) % N_DEV,),
                device_id_type=pl.DeviceIdType.MESH,
            )
        pl.semaphore_wait(barrier_sem, 3)

        xv = x_ref[...]
        psum = jnp.sum(xv, axis=-1)
        psq = jnp.sum(xv * xv, axis=-1)
        own_ref[...] = jnp.concatenate([psum, psq], axis=0)

        rdmas = []
        for d in (1, 2, 3):
            rdma = pltpu.make_async_remote_copy(
                src_ref=own_ref,
                dst_ref=comm_ref.at[3 - d],
                send_sem=send_sems.at[d - 1],
                recv_sem=recv_sems.at[3 - d],
                device_id=((my + d) % N_DEV,),
                device_id_type=pl.DeviceIdType.MESH,
            )
            rdma.start()
            rdmas.append(rdma)

        t = t_ref[...]
        scale = jnp.dot(t, ws_ref[...], preferred_element_type=jnp.float32)
        shift = jnp.dot(t, wsh_ref[...], preferred_element_type=jnp.float32)

        for rdma in rdmas:
            rdma.wait_recv()

        tot = (own_ref[...] + comm_ref[0] + comm_ref[1] + comm_ref[2])
        mean = tot[:b] / C_GLOBAL
        var = tot[b:] / C_GLOBAL - mean * mean
        inv = lax.rsqrt(var + EPS)
        xb = xv.astype(jnp.bfloat16)
        mean_b = mean.astype(jnp.bfloat16)[:, :, None]
        inv_b = inv.astype(jnp.bfloat16)[:, :, None]
        sc_b = (1.0 + scale).astype(jnp.bfloat16)[:, None, :]
        sh_b = shift.astype(jnp.bfloat16)[:, None, :]
        out_ref[...] = ((xb - mean_b) * inv_b) * sc_b + sh_b

        for rdma in rdmas:
            rdma.wait_send()

    return pl.pallas_call(
        body,
        out_shape=jax.ShapeDtypeStruct((b, s, c_loc), jnp.bfloat16),
        in_specs=[pl.BlockSpec(memory_space=pltpu.VMEM)] * 4,
        out_specs=pl.BlockSpec(memory_space=pltpu.VMEM),
        scratch_shapes=[
            pltpu.VMEM((2 * b, s), jnp.float32),
            pltpu.VMEM((3, 2 * b, s), jnp.float32),
            pltpu.SemaphoreType.DMA((3,)),
            pltpu.SemaphoreType.DMA((3,)),
        ],
        compiler_params=pltpu.CompilerParams(collective_id=0),
    )(x, t_emb, W_scale, W_shift)
